# Initial kernel scaffold; baseline (speedup 1.0000x reference)
import jax
import jax.numpy as jnp
from jax.experimental import pallas as pl


def kernel(image, importance):
    raise NotImplementedError("write your pallas kernel here")


# speedup vs baseline: 31.0770x; 31.0770x over previous
import jax
import jax.numpy as jnp
import numpy as np
from jax import lax
from jax.experimental import pallas as pl
from jax.experimental.pallas import tpu as pltpu

_MASK_RATIO = 0.75
_INT_MIN = np.int32(-2147483648)


def _select_body(bits_ref, out_ref, *, len_keep):
    n, hw = bits_ref.shape
    b = bits_ref[...]
    b = jnp.where(b == _INT_MIN, np.int32(0), b)
    ks = b ^ ((b >> 31) & np.int32(0x7FFFFFFF))
    ku = ks ^ _INT_MIN

    def vstep(i, carry):
        pref, rem = carry
        bitpos = 31 - i
        cnt0 = jnp.sum(
            (lax.shift_right_logical(ku, bitpos)
             == lax.shift_right_logical(pref, bitpos)).astype(jnp.int32),
            axis=1, keepdims=True)
        take1 = rem > cnt0
        pref = jnp.where(take1, pref | (np.int32(1) << bitpos), pref)
        rem = jnp.where(take1, rem - cnt0, rem)
        return pref, rem

    pref0 = jnp.zeros((n, 1), jnp.int32)
    rem0 = jnp.full((n, 1), len_keep, jnp.int32)
    pref, rem = lax.fori_loop(0, 32, vstep, (pref0, rem0))

    t_ks = pref ^ _INT_MIN
    eq = ks == t_ks
    lt = ks < t_ks
    idx = lax.broadcasted_iota(jnp.int32, (n, hw), 1)

    def istep(i, carry):
        ipref, irem = carry
        bitpos = 15 - i
        cnt0 = jnp.sum(
            (eq & (lax.shift_right_logical(idx, bitpos)
                   == lax.shift_right_logical(ipref, bitpos))).astype(jnp.int32),
            axis=1, keepdims=True)
        take1 = irem > cnt0
        ipref = jnp.where(take1, ipref | (np.int32(1) << bitpos), ipref)
        irem = jnp.where(take1, irem - cnt0, irem)
        return ipref, irem

    ipref0 = jnp.zeros((n, 1), jnp.int32)
    ipref, _ = lax.fori_loop(0, 16, istep, (ipref0, rem))

    zero = lt | (eq & (idx <= ipref))
    out_ref[...] = 1.0 - zero.astype(jnp.float32)


def kernel(image, importance):
    n, c, h, w = image.shape
    hw = h * w
    len_keep = int(hw * (1 - _MASK_RATIO))
    bits = lax.bitcast_convert_type(importance.reshape(n, hw), jnp.int32)
    import functools
    body = functools.partial(_select_body, len_keep=len_keep)
    mask = pl.pallas_call(
        body,
        out_shape=jax.ShapeDtypeStruct((n, hw), jnp.float32),
    )(bits)
    return mask.reshape(n, 1, h, w)
